# baseline (speedup 1.0000x reference)
import functools

import jax
import jax.numpy as jnp
from jax.experimental import pallas as pl
from jax.experimental.pallas import tpu as pltpu

D = 256


def _transform_body(nparts, nw, *refs):
    parts = refs[:nparts]
    ws = refs[nparts:nparts + nw]
    outs = refs[nparts + nw:]
    x = parts[0][...]
    for p in parts[1:]:
        x = x + p[...]
    if nparts > 1:
        x = x * (1.0 / nparts)
    for w_ref, o_ref in zip(ws, outs):
        o_ref[...] = jnp.dot(x, w_ref[...], preferred_element_type=jnp.float32)


def _transform(parts, ws, bm=512):
    n = parts[0].shape[0]
    bm = min(bm, n)
    in_specs = [pl.BlockSpec((bm, D), lambda i: (i, 0)) for _ in parts]
    in_specs += [pl.BlockSpec((D, D), lambda i: (0, 0)) for _ in ws]
    out_specs = [pl.BlockSpec((bm, D), lambda i: (i, 0)) for _ in ws]
    return pl.pallas_call(
        functools.partial(_transform_body, len(parts), len(ws)),
        grid=(n // bm,),
        in_specs=in_specs,
        out_specs=out_specs,
        out_shape=[jax.ShapeDtypeStruct((n, D), jnp.float32) for _ in ws],
    )(*parts, *ws)


def _mean_body(*refs):
    k = len(refs) - 1
    acc = refs[0][...]
    for r in refs[1:k]:
        acc = acc + r[...]
    refs[k][...] = acc * (1.0 / k)


def _mean(parts, bm=512):
    n = parts[0].shape[0]
    bm = min(bm, n)
    return pl.pallas_call(
        _mean_body,
        grid=(n // bm,),
        in_specs=[pl.BlockSpec((bm, D), lambda i: (i, 0)) for _ in parts],
        out_specs=pl.BlockSpec((bm, D), lambda i: (i, 0)),
        out_shape=jax.ShapeDtypeStruct((n, D), jnp.float32),
    )(*parts)


def _hbs_body(a_ref, c_ref, h_ref, o_ref):
    na = a_ref[...] * c_ref[...]
    o_ref[...] = jnp.dot(na, h_ref[...], preferred_element_type=jnp.float32)


def _hbs(a, c, h, bm=256):
    m, k = a.shape
    bm = min(bm, m)
    return pl.pallas_call(
        _hbs_body,
        grid=(m // bm,),
        in_specs=[
            pl.BlockSpec((bm, k), lambda i: (i, 0)),
            pl.BlockSpec((bm, k), lambda i: (i, 0)),
            pl.BlockSpec((k, D), lambda i: (0, 0)),
        ],
        out_specs=pl.BlockSpec((bm, D), lambda i: (i, 0)),
        out_shape=jax.ShapeDtypeStruct((m, D), jnp.float32),
    )(a, c, h)


def _dual_body(a_ref, c_ref, hs_ref, ht_ref, ot_ref, os_ref):
    i = pl.program_id(0)
    na = a_ref[...] * c_ref[...]
    ot_ref[...] = jnp.dot(na, hs_ref[...], preferred_element_type=jnp.float32)
    contrib = jax.lax.dot_general(
        na, ht_ref[...], (((0,), (0,)), ((), ())),
        preferred_element_type=jnp.float32)
    @pl.when(i == 0)
    def _():
        os_ref[...] = contrib
    @pl.when(i > 0)
    def _():
        os_ref[...] += contrib


def _dual(a, c, hs, ht, bm=256):
    m, k = a.shape
    bm = min(bm, m)
    return pl.pallas_call(
        _dual_body,
        grid=(m // bm,),
        in_specs=[
            pl.BlockSpec((bm, k), lambda i: (i, 0)),
            pl.BlockSpec((bm, k), lambda i: (i, 0)),
            pl.BlockSpec((k, D), lambda i: (0, 0)),
            pl.BlockSpec((bm, D), lambda i: (i, 0)),
        ],
        out_specs=[
            pl.BlockSpec((bm, D), lambda i: (i, 0)),
            pl.BlockSpec((k, D), lambda i: (0, 0)),
        ],
        out_shape=[
            jax.ShapeDtypeStruct((m, D), jnp.float32),
            jax.ShapeDtypeStruct((k, D), jnp.float32),
        ],
        compiler_params=pltpu.CompilerParams(
            dimension_semantics=("arbitrary",)),
    )(a, c, hs, ht)


def kernel(x_0, x_1, x_2, x_3, x_4, adjacency_0, adjacency_1, adjacency_2, adjacency_3, adjacency_4, cci_0_to_0, cci_1_to_1, cci_2_to_2, cci_3_to_3, cci_4_to_4, incidence_0_1, cci_0_to_1, incidence_0_2, cci_0_to_2, incidence_0_3, cci_0_to_3, incidence_0_4, cci_0_to_4, incidence_1_2, cci_1_to_2, incidence_1_3, cci_1_to_3, incidence_1_4, cci_1_to_4, incidence_2_3, cci_2_to_3, incidence_2_4, cci_2_to_4, incidence_3_4, cci_3_to_4, w_hbs_0_l1, w_hbs_4_l1, ws_hbns_0_1_l1, wt_hbns_0_1_l1, ws_hbns_1_2_l1, wt_hbns_1_2_l1, ws_hbns_2_3_l1, wt_hbns_2_3_l1, ws_hbns_3_4_l1, wt_hbns_3_4_l1, w_hbs_0_l2, w_hbs_1_l2, w_hbs_2_l2, w_hbs_3_l2, w_hbs_4_l2, ws_hbns_0_1_l2, wt_hbns_0_1_l2, ws_hbns_1_2_l2, wt_hbns_1_2_l2, ws_hbns_2_3_l2, wt_hbns_2_3_l2, ws_hbns_3_4_l2, wt_hbns_3_4_l2):
    h0_hbs, ft01 = _transform([x_0], [w_hbs_0_l1, wt_hbns_0_1_l1])
    fs01, ft12 = _transform([x_1], [ws_hbns_0_1_l1, wt_hbns_1_2_l1])
    fs12, ft23 = _transform([x_2], [ws_hbns_1_2_l1, wt_hbns_2_3_l1])
    fs23, ft34 = _transform([x_3], [ws_hbns_2_3_l1, wt_hbns_3_4_l1])
    h4_hbs, fs34 = _transform([x_4], [w_hbs_4_l1, ws_hbns_3_4_l1])

    x00 = _hbs(adjacency_0, cci_0_to_0, h0_hbs)
    x44 = _hbs(adjacency_4, cci_4_to_4, h4_hbs)
    x10, x01 = _dual(incidence_0_1, cci_0_to_1, fs01, ft01)
    x21, x12 = _dual(incidence_1_2, cci_1_to_2, fs12, ft12)
    x32, x23 = _dual(incidence_2_3, cci_2_to_3, fs23, ft23)
    x43, x34 = _dual(incidence_3_4, cci_3_to_4, fs34, ft34)

    h0b, ft01b = _transform([x00, x10], [w_hbs_0_l2, wt_hbns_0_1_l2])
    h1b, fs01b, ft12b = _transform([x01, x21], [w_hbs_1_l2, ws_hbns_0_1_l2, wt_hbns_1_2_l2])
    h2b, fs12b, ft23b = _transform([x12, x32], [w_hbs_2_l2, ws_hbns_1_2_l2, wt_hbns_2_3_l2])
    h3b, fs23b, ft34b = _transform([x23, x43], [w_hbs_3_l2, ws_hbns_2_3_l2, wt_hbns_3_4_l2])
    h4b, fs34b = _transform([x34, x44], [w_hbs_4_l2, ws_hbns_3_4_l2])

    y00 = _hbs(adjacency_0, cci_0_to_0, h0b)
    y11 = _hbs(adjacency_1, cci_1_to_1, h1b)
    y22 = _hbs(adjacency_2, cci_2_to_2, h2b)
    y33 = _hbs(adjacency_3, cci_3_to_3, h3b)
    y44 = _hbs(adjacency_4, cci_4_to_4, h4b)
    y10, y01 = _dual(incidence_0_1, cci_0_to_1, fs01b, ft01b)
    y21, y12 = _dual(incidence_1_2, cci_1_to_2, fs12b, ft12b)
    y32, y23 = _dual(incidence_2_3, cci_2_to_3, fs23b, ft23b)
    y43, y34 = _dual(incidence_3_4, cci_3_to_4, fs34b, ft34b)

    out0 = _mean([y00, y10])
    out1 = _mean([y01, y11, y21])
    out2 = _mean([y12, y22, y32])
    out3 = _mean([y23, y33, y43])
    out4 = _mean([y34, y44])
    return (out0, out1, out2, out3, out4)


# speedup vs baseline: 1.0807x; 1.0807x over previous
import functools

import jax
import jax.numpy as jnp
from jax.experimental import pallas as pl
from jax.experimental.pallas import tpu as pltpu

D = 256


def _transform_body(nparts, nw, *refs):
    parts = refs[:nparts]
    ws = refs[nparts:nparts + nw]
    outs = refs[nparts + nw:]
    x = parts[0][...]
    for p in parts[1:]:
        x = x + p[...]
    if nparts > 1:
        x = x * (1.0 / nparts)
    for w_ref, o_ref in zip(ws, outs):
        o_ref[...] = jnp.dot(x, w_ref[...], preferred_element_type=jnp.float32)


def _transform(parts, ws, bm=512):
    n = parts[0].shape[0]
    bm = min(bm, n)
    in_specs = [pl.BlockSpec((bm, D), lambda i: (i, 0)) for _ in parts]
    in_specs += [pl.BlockSpec((D, D), lambda i: (0, 0)) for _ in ws]
    out_specs = [pl.BlockSpec((bm, D), lambda i: (i, 0)) for _ in ws]
    return pl.pallas_call(
        functools.partial(_transform_body, len(parts), len(ws)),
        grid=(n // bm,),
        in_specs=in_specs,
        out_specs=out_specs,
        out_shape=[jax.ShapeDtypeStruct((n, D), jnp.float32) for _ in ws],
    )(*parts, *ws)


def _mean_body(*refs):
    k = len(refs) - 1
    acc = refs[0][...]
    for r in refs[1:k]:
        acc = acc + r[...]
    refs[k][...] = acc * (1.0 / k)


def _mean(parts, bm=512):
    n = parts[0].shape[0]
    bm = min(bm, n)
    return pl.pallas_call(
        _mean_body,
        grid=(n // bm,),
        in_specs=[pl.BlockSpec((bm, D), lambda i: (i, 0)) for _ in parts],
        out_specs=pl.BlockSpec((bm, D), lambda i: (i, 0)),
        out_shape=jax.ShapeDtypeStruct((n, D), jnp.float32),
    )(*parts)


def _hbs_body(a_ref, c_ref, h_ref, o_ref):
    na = a_ref[...] * c_ref[...]
    o_ref[...] = jnp.dot(na, h_ref[...], preferred_element_type=jnp.float32)


def _hbs(a, c, h, bm=512):
    m, k = a.shape
    bm = min(bm, m)
    return pl.pallas_call(
        _hbs_body,
        grid=(m // bm,),
        in_specs=[
            pl.BlockSpec((bm, k), lambda i: (i, 0)),
            pl.BlockSpec((bm, k), lambda i: (i, 0)),
            pl.BlockSpec((k, D), lambda i: (0, 0)),
        ],
        out_specs=pl.BlockSpec((bm, D), lambda i: (i, 0)),
        out_shape=jax.ShapeDtypeStruct((m, D), jnp.float32),
    )(a, c, h)


def _dual_body(a_ref, c_ref, hs_ref, ht_ref, ot_ref, os_ref):
    i = pl.program_id(0)
    na = a_ref[...] * c_ref[...]
    ot_ref[...] = jnp.dot(na, hs_ref[...], preferred_element_type=jnp.float32)
    contrib = jax.lax.dot_general(
        na, ht_ref[...], (((0,), (0,)), ((), ())),
        preferred_element_type=jnp.float32)
    @pl.when(i == 0)
    def _():
        os_ref[...] = contrib
    @pl.when(i > 0)
    def _():
        os_ref[...] += contrib


def _dual(a, c, hs, ht, bm=512):
    m, k = a.shape
    bm = min(bm, m)
    return pl.pallas_call(
        _dual_body,
        grid=(m // bm,),
        in_specs=[
            pl.BlockSpec((bm, k), lambda i: (i, 0)),
            pl.BlockSpec((bm, k), lambda i: (i, 0)),
            pl.BlockSpec((k, D), lambda i: (0, 0)),
            pl.BlockSpec((bm, D), lambda i: (i, 0)),
        ],
        out_specs=[
            pl.BlockSpec((bm, D), lambda i: (i, 0)),
            pl.BlockSpec((k, D), lambda i: (0, 0)),
        ],
        out_shape=[
            jax.ShapeDtypeStruct((m, D), jnp.float32),
            jax.ShapeDtypeStruct((k, D), jnp.float32),
        ],
        compiler_params=pltpu.CompilerParams(
            dimension_semantics=("arbitrary",)),
    )(a, c, hs, ht)


def kernel(x_0, x_1, x_2, x_3, x_4, adjacency_0, adjacency_1, adjacency_2, adjacency_3, adjacency_4, cci_0_to_0, cci_1_to_1, cci_2_to_2, cci_3_to_3, cci_4_to_4, incidence_0_1, cci_0_to_1, incidence_0_2, cci_0_to_2, incidence_0_3, cci_0_to_3, incidence_0_4, cci_0_to_4, incidence_1_2, cci_1_to_2, incidence_1_3, cci_1_to_3, incidence_1_4, cci_1_to_4, incidence_2_3, cci_2_to_3, incidence_2_4, cci_2_to_4, incidence_3_4, cci_3_to_4, w_hbs_0_l1, w_hbs_4_l1, ws_hbns_0_1_l1, wt_hbns_0_1_l1, ws_hbns_1_2_l1, wt_hbns_1_2_l1, ws_hbns_2_3_l1, wt_hbns_2_3_l1, ws_hbns_3_4_l1, wt_hbns_3_4_l1, w_hbs_0_l2, w_hbs_1_l2, w_hbs_2_l2, w_hbs_3_l2, w_hbs_4_l2, ws_hbns_0_1_l2, wt_hbns_0_1_l2, ws_hbns_1_2_l2, wt_hbns_1_2_l2, ws_hbns_2_3_l2, wt_hbns_2_3_l2, ws_hbns_3_4_l2, wt_hbns_3_4_l2):
    h0_hbs, ft01 = _transform([x_0], [w_hbs_0_l1, wt_hbns_0_1_l1])
    fs01, ft12 = _transform([x_1], [ws_hbns_0_1_l1, wt_hbns_1_2_l1])
    fs12, ft23 = _transform([x_2], [ws_hbns_1_2_l1, wt_hbns_2_3_l1])
    fs23, ft34 = _transform([x_3], [ws_hbns_2_3_l1, wt_hbns_3_4_l1])
    h4_hbs, fs34 = _transform([x_4], [w_hbs_4_l1, ws_hbns_3_4_l1])

    x00 = _hbs(adjacency_0, cci_0_to_0, h0_hbs)
    x44 = _hbs(adjacency_4, cci_4_to_4, h4_hbs)
    x10, x01 = _dual(incidence_0_1, cci_0_to_1, fs01, ft01)
    x21, x12 = _dual(incidence_1_2, cci_1_to_2, fs12, ft12)
    x32, x23 = _dual(incidence_2_3, cci_2_to_3, fs23, ft23)
    x43, x34 = _dual(incidence_3_4, cci_3_to_4, fs34, ft34)

    h0b, ft01b = _transform([x00, x10], [w_hbs_0_l2, wt_hbns_0_1_l2])
    h1b, fs01b, ft12b = _transform([x01, x21], [w_hbs_1_l2, ws_hbns_0_1_l2, wt_hbns_1_2_l2])
    h2b, fs12b, ft23b = _transform([x12, x32], [w_hbs_2_l2, ws_hbns_1_2_l2, wt_hbns_2_3_l2])
    h3b, fs23b, ft34b = _transform([x23, x43], [w_hbs_3_l2, ws_hbns_2_3_l2, wt_hbns_3_4_l2])
    h4b, fs34b = _transform([x34, x44], [w_hbs_4_l2, ws_hbns_3_4_l2])

    y00 = _hbs(adjacency_0, cci_0_to_0, h0b)
    y11 = _hbs(adjacency_1, cci_1_to_1, h1b)
    y22 = _hbs(adjacency_2, cci_2_to_2, h2b)
    y33 = _hbs(adjacency_3, cci_3_to_3, h3b)
    y44 = _hbs(adjacency_4, cci_4_to_4, h4b)
    y10, y01 = _dual(incidence_0_1, cci_0_to_1, fs01b, ft01b)
    y21, y12 = _dual(incidence_1_2, cci_1_to_2, fs12b, ft12b)
    y32, y23 = _dual(incidence_2_3, cci_2_to_3, fs23b, ft23b)
    y43, y34 = _dual(incidence_3_4, cci_3_to_4, fs34b, ft34b)

    out0 = _mean([y00, y10])
    out1 = _mean([y01, y11, y21])
    out2 = _mean([y12, y22, y32])
    out3 = _mean([y23, y33, y43])
    out4 = _mean([y34, y44])
    return (out0, out1, out2, out3, out4)
